# baseline (speedup 1.0000x reference)
import jax
import jax.numpy as jnp
from jax.experimental import pallas as pl
from jax.experimental.pallas import tpu as pltpu


def _probe_body(x_ref, o_ref):
    o_ref[...] = jnp.sum(x_ref[...], axis=(1, 2), keepdims=True)


def kernel(x, w1, b1, w2, b2):
    B, C, H, W = x.shape
    HW = H * W
    R = C * HW // 8
    x_v = x.reshape(B, 8, R)
    out = pl.pallas_call(
        _probe_body,
        out_shape=jax.ShapeDtypeStruct((B, 1, 1), jnp.float32),
        grid=(B,),
        in_specs=[pl.BlockSpec((1, 8, R), lambda b: (b, 0, 0))],
        out_specs=pl.BlockSpec((1, 1, 1), lambda b: (b, 0, 0)),
        compiler_params=pltpu.CompilerParams(
            dimension_semantics=("parallel",),
            vmem_limit_bytes=int(64 * 1024 * 1024 * 0.9),
        ),
    )(x_v)
    return out.reshape(B, 1, 1, 1).astype(x.dtype)


# speedup vs baseline: 4.2918x; 4.2918x over previous
import functools

import jax
import jax.numpy as jnp
from jax.experimental import pallas as pl
from jax.experimental.pallas import tpu as pltpu


def _ca_tiled_body(x_ref, w1jt_ref, b1_ref, w2jt_ref, b2_ref, o_ref, *,
                   inv_hw, c128):
    xb = x_ref[0]
    n = xb.shape[1]
    xsum = jnp.sum(xb, axis=0, dtype=jnp.float32)
    pooled = jnp.sum(xsum.reshape(c128, n // c128, 128),
                     axis=1) * inv_hw

    h = b1_ref[...]
    for j in range(c128):
        h = h + jnp.dot(pooled[j:j + 1, :], w1jt_ref[j],
                        preferred_element_type=jnp.float32)
    h = jnp.maximum(h, 0.0)

    srows = [
        jax.nn.sigmoid(jnp.dot(h, w2jt_ref[j],
                               preferred_element_type=jnp.float32)
                       + b2_ref[j:j + 1, :])
        for j in range(c128)
    ]
    s2 = jnp.concatenate(srows, axis=0)
    sfull = jnp.broadcast_to(
        s2[:, None, :], (c128, n // c128, 128)).reshape(n, 128)
    o_ref[0] = xb * sfull[None]


def _ca_flat_body(x_ref, w1_ref, b1_ref, w2_ref, b2_ref, o_ref, *, inv_hw):
    x = x_ref[0]
    pooled = jnp.sum(x, axis=-1, keepdims=True,
                     dtype=jnp.float32) * inv_hw
    h = jnp.dot(w1_ref[...], pooled,
                preferred_element_type=jnp.float32) + b1_ref[...]
    h = jnp.maximum(h, 0.0)
    z = jnp.dot(w2_ref[...], h,
                preferred_element_type=jnp.float32) + b2_ref[...]
    s = jax.nn.sigmoid(z)
    o_ref[0] = x * s


def _vmem_limit():
    return int(64 * 1024 * 1024 * 0.9)


def kernel(x, w1, b1, w2, b2):
    B, C, H, W = x.shape
    Cr = w1.shape[0]
    HW = H * W
    inv_hw = float(1.0 / HW)
    itemsize = jnp.dtype(x.dtype).itemsize
    cost = pl.CostEstimate(
        flops=int(B * (2 * C * HW + 4 * C * Cr)),
        transcendentals=int(B * C),
        bytes_accessed=int(2 * B * C * HW * itemsize),
    )

    if W % 8 == 0 and C % 128 == 0 and x.dtype == jnp.float32:
        W8, C128 = W // 8, C // 128
        M, N = H * W8, C128 * 8
        xv = (x.transpose(0, 2, 3, 1)
               .reshape(B, H, W8, 8, C128, 128)
               .transpose(0, 1, 2, 4, 3, 5)
               .reshape(B, M, N, 128))

        w1r = w1.reshape(Cr, C).astype(jnp.float32)
        w1jt = w1r.reshape(Cr, C128, 128).transpose(1, 2, 0)
        b1r = b1.astype(jnp.float32).reshape(1, Cr)
        w2r = w2.reshape(C, Cr).astype(jnp.float32)
        w2jt = w2r.reshape(C128, 128, Cr).transpose(0, 2, 1)
        b2r = b2.astype(jnp.float32).reshape(C128, 128)

        body = functools.partial(_ca_tiled_body, inv_hw=inv_hw, c128=C128)
        out_v = pl.pallas_call(
            body,
            out_shape=jax.ShapeDtypeStruct((B, M, N, 128), x.dtype),
            grid=(B,),
            in_specs=[
                pl.BlockSpec((1, M, N, 128), lambda b: (b, 0, 0, 0)),
                pl.BlockSpec((C128, 128, Cr), lambda b: (0, 0, 0)),
                pl.BlockSpec((1, Cr), lambda b: (0, 0)),
                pl.BlockSpec((C128, Cr, 128), lambda b: (0, 0, 0)),
                pl.BlockSpec((C128, 128), lambda b: (0, 0)),
            ],
            out_specs=pl.BlockSpec((1, M, N, 128), lambda b: (b, 0, 0, 0)),
            compiler_params=pltpu.CompilerParams(
                dimension_semantics=("parallel",),
                vmem_limit_bytes=_vmem_limit(),
            ),
            cost_estimate=cost,
        )(xv, w1jt, b1r, w2jt, b2r)

        out = (out_v.reshape(B, H, W8, C128, 8, 128)
               .transpose(0, 1, 2, 4, 3, 5)
               .reshape(B, H, W, C)
               .transpose(0, 3, 1, 2))
        return out

    x_flat = x.reshape(B, C, HW)
    w1m = w1.reshape(Cr, C).astype(jnp.float32)
    b1c = b1.astype(jnp.float32).reshape(Cr, 1)
    w2m = w2.reshape(C, Cr).astype(jnp.float32)
    b2c = b2.astype(jnp.float32).reshape(C, 1)
    out_flat = pl.pallas_call(
        functools.partial(_ca_flat_body, inv_hw=inv_hw),
        out_shape=jax.ShapeDtypeStruct((B, C, HW), x.dtype),
        grid=(B,),
        in_specs=[
            pl.BlockSpec((1, C, HW), lambda b: (b, 0, 0)),
            pl.BlockSpec((Cr, C), lambda b: (0, 0)),
            pl.BlockSpec((Cr, 1), lambda b: (0, 0)),
            pl.BlockSpec((C, Cr), lambda b: (0, 0)),
            pl.BlockSpec((C, 1), lambda b: (0, 0)),
        ],
        out_specs=pl.BlockSpec((1, C, HW), lambda b: (b, 0, 0)),
        compiler_params=pltpu.CompilerParams(
            dimension_semantics=("parallel",),
            vmem_limit_bytes=_vmem_limit(),
        ),
        cost_estimate=cost,
    )(x_flat, w1m, b1c, w2m, b2c)
    return out_flat.reshape(B, C, H, W)
